# baseline (speedup 1.0000x reference)
import functools

import jax
import jax.numpy as jnp
from jax import lax
from jax.experimental import pallas as pl
from jax.experimental.pallas import tpu as pltpu
from jax.experimental.pallas import tpu_sc as plsc

LQ = 2048
LK = 2048
D = 32
MAX_REL = 128
ROWS = 2 * MAX_REL + 1

_info = plsc.get_sparse_core_info()
NC = _info.num_cores
NS = _info.num_subcores
NW = NC * NS
QPW = LQ // NW
BAND = LK + QPW


@functools.partial(
    pl.kernel,
    mesh=plsc.VectorSubcoreMesh(core_axis_name="c", subcore_axis_name="s"),
    compiler_params=pltpu.CompilerParams(use_tc_tiling_on_sc=False),
    out_type=jax.ShapeDtypeStruct((LQ, LK, D), jnp.float32),
    scratch_types=[
        pltpu.VMEM((ROWS, D), jnp.float32),
        pltpu.VMEM((BAND, D), jnp.float32),
        pltpu.SemaphoreType.DMA,
    ],
)
def _rel_pos_sc(table_hbm, out_hbm, table_v, band_v, sem):
    wid = lax.axis_index("s") * NC + lax.axis_index("c")
    q0 = wid * QPW
    g0 = (LK - 1) - (q0 + QPW - 1)

    pltpu.sync_copy(table_hbm, table_v)

    def build_row(j, carry):
        t = jnp.clip(g0 + j - (LK - 1), -MAX_REL, MAX_REL) + MAX_REL
        band_v[j, pl.ds(0, 16)] = table_v[t, pl.ds(0, 16)]
        band_v[j, pl.ds(16, 16)] = table_v[t, pl.ds(16, 16)]
        return carry

    lax.fori_loop(0, BAND, build_row, 0)

    def fire_row(r, carry):
        pltpu.async_copy(
            band_v.at[pl.ds((QPW - 1) - r, LK), :],
            out_hbm.at[q0 + r],
            sem,
        )
        return carry

    lax.fori_loop(0, QPW, fire_row, 0)

    def drain_row(r, carry):
        pltpu.make_async_copy(
            band_v.at[pl.ds(0, LK), :],
            out_hbm.at[q0],
            sem,
        ).wait()
        return carry

    lax.fori_loop(0, QPW, drain_row, 0)


def kernel(length_q, length_k, embeddings_table):
    del length_q, length_k
    return _rel_pos_sc(embeddings_table)


# speedup vs baseline: 9.7502x; 9.7502x over previous
import functools

import jax
import jax.numpy as jnp
from jax import lax
from jax.experimental import pallas as pl
from jax.experimental.pallas import tpu as pltpu
from jax.experimental.pallas import tpu_sc as plsc

LQ = 2048
LK = 2048
D = 32
MAX_REL = 128
ROWS = 2 * MAX_REL + 1

_info = plsc.get_sparse_core_info()
NC = _info.num_cores
NS = _info.num_subcores
NW = NC * NS
QPW = LQ // NW

ND8 = D // 8
NK128 = LK // 128
EW = 768
JMAX = EW - 128


@functools.partial(
    pl.kernel,
    mesh=plsc.VectorSubcoreMesh(core_axis_name="c", subcore_axis_name="s"),
    compiler_params=pltpu.CompilerParams(use_tc_tiling_on_sc=False),
    out_type=jax.ShapeDtypeStruct((LQ, ND8, NK128, 8, 128), jnp.float32),
    scratch_types=[
        pltpu.VMEM((D, EW), jnp.float32),
        pltpu.SemaphoreType.DMA,
    ],
)
def _rel_pos_sc(te8_hbm, out_hbm, te_v, sem):
    wid = lax.axis_index("s") * NC + lax.axis_index("c")
    c = wid % 8
    b = wid // 8
    p = (383 - c) % 8

    pltpu.sync_copy(te8_hbm.at[p], te_v)

    def row_step(i, carry):
        q = c + 8 * (QPW * b + i)
        for k in range(NK128):
            j0 = jnp.clip(128 * k - q + 383 - p, 0, JMAX)
            j0 = pl.multiple_of(j0, 8)
            for d8 in range(ND8):
                pltpu.async_copy(
                    te_v.at[pl.ds(8 * d8, 8), pl.ds(j0, 128)],
                    out_hbm.at[q, d8, k],
                    sem,
                )

        @pl.when(i > 0)
        def _():
            def drain(_, cc):
                pltpu.make_async_copy(
                    te_v.at[pl.ds(0, 8), pl.ds(0, 128)],
                    out_hbm.at[q, 0, 0],
                    sem,
                ).wait()
                return cc

            lax.fori_loop(0, NK128 * ND8, drain, 0)

        return carry

    lax.fori_loop(0, QPW, row_step, 0)

    def drain_last(_, cc):
        pltpu.make_async_copy(
            te_v.at[pl.ds(0, 8), pl.ds(0, 128)],
            out_hbm.at[c, 0, 0],
            sem,
        ).wait()
        return cc

    lax.fori_loop(0, NK128 * ND8, drain_last, 0)


def kernel(length_q, length_k, embeddings_table):
    del length_q, length_k
    j = jnp.arange(EW, dtype=jnp.int32)[None, :] + jnp.arange(8, dtype=jnp.int32)[:, None]
    rows = jnp.clip(j - 383, -MAX_REL, MAX_REL) + MAX_REL
    te8 = jnp.transpose(embeddings_table[rows, :], (0, 2, 1))
    y5 = _rel_pos_sc(te8)
    return y5.transpose(0, 2, 4, 1, 3).reshape(LQ, LK, D)


# speedup vs baseline: 9.7853x; 1.0036x over previous
import functools

import jax
import jax.numpy as jnp
from jax import lax
from jax.experimental import pallas as pl
from jax.experimental.pallas import tpu as pltpu
from jax.experimental.pallas import tpu_sc as plsc

LQ = 2048
LK = 2048
D = 32
MAX_REL = 128
ROWS = 2 * MAX_REL + 1

_info = plsc.get_sparse_core_info()
NC = _info.num_cores
NS = _info.num_subcores
NW = NC * NS
QPW = LQ // NW

ND8 = D // 8
NK128 = LK // 128
EW = 768
JMAX = EW - 128


@functools.partial(
    pl.kernel,
    mesh=plsc.VectorSubcoreMesh(core_axis_name="c", subcore_axis_name="s"),
    compiler_params=pltpu.CompilerParams(use_tc_tiling_on_sc=False),
    out_type=jax.ShapeDtypeStruct((LQ, ND8, NK128, 8, 128), jnp.float32),
    scratch_types=[
        pltpu.VMEM((ND8, 1, 8, EW), jnp.float32),
        pltpu.SemaphoreType.DMA,
    ],
)
def _rel_pos_sc(te8_hbm, out_hbm, te_v, sem):
    wid = lax.axis_index("s") * NC + lax.axis_index("c")
    c = wid % 8
    b = wid // 8
    p = (383 - c) % 8

    pltpu.sync_copy(te8_hbm.at[p], te_v)

    def row_step(i, carry):
        q = c + 8 * (QPW * b + i)
        for k in range(NK128):
            j0 = jnp.clip(128 * k - q + 383 - p, 0, JMAX)
            j0 = pl.multiple_of(j0, 8)
            pltpu.async_copy(
                te_v.at[:, :, :, pl.ds(j0, 128)],
                out_hbm.at[q, pl.ds(0, ND8), pl.ds(k, 1)],
                sem,
            )

        @pl.when(i > 0)
        def _():
            def drain(_, cc):
                pltpu.make_async_copy(
                    te_v.at[:, :, :, pl.ds(0, 128)],
                    out_hbm.at[q, pl.ds(0, ND8), pl.ds(0, 1)],
                    sem,
                ).wait()
                return cc

            lax.fori_loop(0, NK128, drain, 0)

        return carry

    lax.fori_loop(0, QPW, row_step, 0)

    def drain_last(_, cc):
        pltpu.make_async_copy(
            te_v.at[:, :, :, pl.ds(0, 128)],
            out_hbm.at[c, pl.ds(0, ND8), pl.ds(0, 1)],
            sem,
        ).wait()
        return cc

    lax.fori_loop(0, NK128, drain_last, 0)


def kernel(length_q, length_k, embeddings_table):
    del length_q, length_k
    j = jnp.arange(EW, dtype=jnp.int32)[None, :] + jnp.arange(8, dtype=jnp.int32)[:, None]
    rows = jnp.clip(j - 383, -MAX_REL, MAX_REL) + MAX_REL
    te8 = jnp.transpose(embeddings_table[rows, :], (0, 2, 1))
    te8 = te8.reshape(8, ND8, 1, 8, EW)
    y5 = _rel_pos_sc(te8)
    return y5.transpose(0, 2, 4, 1, 3).reshape(LQ, LK, D)


# speedup vs baseline: 9.8112x; 1.0026x over previous
import functools

import jax
import jax.numpy as jnp
from jax import lax
from jax.experimental import pallas as pl
from jax.experimental.pallas import tpu as pltpu
from jax.experimental.pallas import tpu_sc as plsc

LQ = 2048
LK = 2048
D = 32
MAX_REL = 128

_info = plsc.get_sparse_core_info()
NC = _info.num_cores
NS = _info.num_subcores
NW = NC * NS
QPW = LQ // NW

ND8 = D // 8
NK128 = LK // 128
EW = 768
JMAX = EW - 128


@functools.partial(
    pl.kernel,
    mesh=plsc.VectorSubcoreMesh(core_axis_name="c", subcore_axis_name="s"),
    compiler_params=pltpu.CompilerParams(use_tc_tiling_on_sc=False),
    out_type=jax.ShapeDtypeStruct((LQ, ND8, NK128, 8, 128), jnp.float32),
    scratch_types=[
        pltpu.VMEM((ND8, 1, 8, EW), jnp.float32),
        pltpu.SemaphoreType.DMA,
    ],
)
def _rel_pos_sc(te8_hbm, out_hbm, te_v, sem):
    wid = lax.axis_index("s") * NC + lax.axis_index("c")
    c = wid % 8
    b = wid // 8
    p = (383 - c) % 8

    pltpu.sync_copy(te8_hbm.at[p], te_v)

    def row_step(i, carry):
        q = c + 8 * (QPW * b + i)
        for k in range(NK128):
            j0 = jnp.clip(128 * k - q + 383 - p, 0, JMAX)
            j0 = pl.multiple_of(j0, 8)
            pltpu.async_copy(
                te_v.at[:, :, :, pl.ds(j0, 128)],
                out_hbm.at[q, pl.ds(0, ND8), pl.ds(k, 1)],
                sem,
            )

        @pl.when(i > 0)
        def _():
            def drain(_, cc):
                pltpu.make_async_copy(
                    te_v.at[:, :, :, pl.ds(0, 128)],
                    out_hbm.at[q, pl.ds(0, ND8), pl.ds(0, 1)],
                    sem,
                ).wait()
                return cc

            lax.fori_loop(0, NK128, drain, 0)

        return carry

    lax.fori_loop(0, QPW, row_step, 0)

    def drain_last(_, cc):
        pltpu.make_async_copy(
            te_v.at[:, :, :, pl.ds(0, 128)],
            out_hbm.at[c, pl.ds(0, ND8), pl.ds(0, 1)],
            sem,
        ).wait()
        return cc

    lax.fori_loop(0, NK128, drain_last, 0)


def kernel(length_q, length_k, embeddings_table):
    del length_q, length_k
    j = jnp.arange(EW, dtype=jnp.int32)[None, :] + jnp.arange(8, dtype=jnp.int32)[:, None]
    rows = jnp.clip(j - 383, -MAX_REL, MAX_REL) + MAX_REL
    te8 = jnp.transpose(embeddings_table[rows, :], (0, 2, 1))
    te8 = te8.reshape(8, ND8, 1, 8, EW)
    y5 = _rel_pos_sc(te8)
    return y5.transpose(0, 2, 4, 1, 3).reshape(LQ, LK, D)
